# Initial kernel scaffold; baseline (speedup 1.0000x reference)
import functools

import jax
import jax.numpy as jnp
from jax import lax
from jax.experimental import pallas as pl
from jax.experimental.pallas import tpu as pltpu
from jax.experimental.pallas import tpu_sc as plsc

NV = 10000
NE = 5000
NEP = 5008
NNZ = 320000
CHUNK = 8000
NGRP = CHUNK // 16
NCHUNK = NNZ // CHUNK
NC = 2
NS = 16
NW = NC * NS


def _mm_body(w_ref, x_ref, b_ref, o_ref, *, dims):
    o_ref[...] = lax.dot_general(
        w_ref[...], x_ref[...], dims, preferred_element_type=jnp.float32
    ) + b_ref[...]


def _mm_xt(W, X, b):
    do = W.shape[1]
    return pl.pallas_call(
        functools.partial(_mm_body, dims=(((0,), (1,)), ((), ()))),
        out_shape=jax.ShapeDtypeStruct((do, X.shape[0]), jnp.float32),
    )(W, X, b.reshape(do, 1))


def _mm_tt(W, Zt, b):
    do = W.shape[1]
    return pl.pallas_call(
        functools.partial(_mm_body, dims=(((0,), (0,)), ((), ()))),
        out_shape=jax.ShapeDtypeStruct((do, Zt.shape[1]), jnp.float32),
    )(W, Zt, b.reshape(do, 1))


def _fill_rows(ref, c_range, n16, val16):
    def body(j, _):
        for c in c_range:
            ref[c, pl.ds(j * 16, 16)] = val16
        return 0
    lax.fori_loop(0, n16, body, 0, unroll=4)


def _sc_body(*refs, C, first, relu):
    if first:
        (yt, vids, eids, zt, rvd_out, red_out,
         a_buf, e_buf, vd, ed, vv, ee) = refs
    else:
        (yt, vids, eids, rvd_in, red_in, zt,
         a_buf, e_buf, vd, ed, vv, ee) = refs

    wid = lax.axis_index("s") * NC + lax.axis_index("c")
    row0 = wid * C

    zeros16 = jnp.zeros((16,), jnp.float32)
    ones16 = jnp.full((16,), 1.0, jnp.float32)
    cvecs = [jnp.full((16,), c, jnp.int32) for c in range(C)]

    pltpu.sync_copy(yt.at[pl.ds(row0, C)], a_buf)

    _fill_rows(e_buf, range(C), NEP // 16, zeros16)
    if first:
        def zdeg(j, _):
            vd[pl.ds(j * 16, 16)] = zeros16
            return 0
        lax.fori_loop(0, NV // 16, zdeg, 0, unroll=4)

        def zdeg_e(j, _):
            ed[pl.ds(j * 16, 16)] = zeros16
            return 0
        lax.fori_loop(0, NEP // 16, zdeg_e, 0, unroll=4)
    else:
        pltpu.sync_copy(rvd_in, vd)
        pltpu.sync_copy(red_in, ed)

    def pass1_chunk(k, _):
        base = pl.multiple_of(k * CHUNK, 8)
        pltpu.sync_copy(vids.at[pl.ds(base, CHUNK)], vv)
        pltpu.sync_copy(eids.at[pl.ds(base, CHUNK)], ee)

        def grp(g, _):
            off = g * 16
            v16 = vv[pl.ds(off, 16)]
            e16 = ee[pl.ds(off, 16)]
            if first:
                plsc.addupdate_scatter(vd, [v16], ones16)
                plsc.addupdate_scatter(ed, [e16], ones16)
            for c in range(C):
                vals = plsc.load_gather(a_buf, [cvecs[c], v16])
                plsc.addupdate_scatter(e_buf, [cvecs[c], e16], vals)
            return 0

        lax.fori_loop(0, NGRP, grp, 0, unroll=4)
        return 0

    lax.fori_loop(0, NCHUNK, pass1_chunk, 0)

    if first:
        def rvd_loop(j, _):
            s = pl.ds(j * 16, 16)
            vd[s] = 1.0 / jnp.maximum(vd[s], 1.0)
            return 0
        lax.fori_loop(0, NV // 16, rvd_loop, 0, unroll=4)

        def red_loop(j, _):
            s = pl.ds(j * 16, 16)
            ed[s] = 1.0 / jnp.maximum(ed[s], 1.0)
            return 0
        lax.fori_loop(0, NEP // 16, red_loop, 0, unroll=4)

    def esc(j, _):
        s = pl.ds(j * 16, 16)
        r = ed[s]
        for c in range(C):
            e_buf[c, s] = e_buf[c, s] * r
        return 0
    lax.fori_loop(0, NEP // 16, esc, 0, unroll=4)

    _fill_rows(a_buf, range(C), NV // 16, zeros16)

    def pass2_chunk(k, _):
        base = pl.multiple_of(k * CHUNK, 8)
        pltpu.sync_copy(vids.at[pl.ds(base, CHUNK)], vv)
        pltpu.sync_copy(eids.at[pl.ds(base, CHUNK)], ee)

        def grp(g, _):
            off = g * 16
            v16 = vv[pl.ds(off, 16)]
            e16 = ee[pl.ds(off, 16)]
            for c in range(C):
                vals = plsc.load_gather(e_buf, [cvecs[c], e16])
                plsc.addupdate_scatter(a_buf, [cvecs[c], v16], vals)
            return 0

        lax.fori_loop(0, NGRP, grp, 0, unroll=4)
        return 0

    lax.fori_loop(0, NCHUNK, pass2_chunk, 0)

    def vsc(j, _):
        s = pl.ds(j * 16, 16)
        r = vd[s]
        for c in range(C):
            x = a_buf[c, s] * r
            if relu:
                x = jnp.maximum(x, 0.0)
            a_buf[c, s] = x
        return 0
    lax.fori_loop(0, NV // 16, vsc, 0, unroll=4)

    pltpu.sync_copy(a_buf, zt.at[pl.ds(row0, C)])

    if first:
        @pl.when(wid == 0)
        def _():
            pltpu.sync_copy(vd, rvd_out)
            pltpu.sync_copy(ed, red_out)


def _make_sc(C, first, relu):
    d = C * NW
    out_type = [jax.ShapeDtypeStruct((d, NV), jnp.float32)]
    if first:
        out_type += [jax.ShapeDtypeStruct((NV,), jnp.float32),
                     jax.ShapeDtypeStruct((NEP,), jnp.float32)]
    scratch = [
        pltpu.VMEM((C, NV), jnp.float32),
        pltpu.VMEM((C, NEP), jnp.float32),
        pltpu.VMEM((NV,), jnp.float32),
        pltpu.VMEM((NEP,), jnp.float32),
        pltpu.VMEM((CHUNK,), jnp.int32),
        pltpu.VMEM((CHUNK,), jnp.int32),
    ]
    mesh = plsc.VectorSubcoreMesh(core_axis_name="c", subcore_axis_name="s")
    return pl.kernel(
        functools.partial(_sc_body, C=C, first=first, relu=relu),
        out_type=out_type,
        mesh=mesh,
        scratch_types=scratch,
    )


def kernel(X, v_ids, e_ids, W0, b0, W1, b1, W2, b2):
    v32 = v_ids.astype(jnp.int32)
    e32 = e_ids.astype(jnp.int32)

    sc_first = _make_sc(4, True, True)
    sc_mid = _make_sc(4, False, True)
    sc_last = _make_sc(2, False, False)

    y0 = _mm_xt(W0, X, b0)
    z0, rvd, red = sc_first(y0, v32, e32)
    y1 = _mm_tt(W1, z0, b1)
    z1 = sc_mid(y1, v32, e32, rvd, red)
    y2 = _mm_tt(W2, z1, b2)
    z2 = sc_last(y2, v32, e32, rvd, red)
    return z2.T


# speedup vs baseline: 2.9443x; 2.9443x over previous
import functools

import jax
import jax.numpy as jnp
from jax import lax
from jax.experimental import pallas as pl
from jax.experimental.pallas import tpu as pltpu
from jax.experimental.pallas import tpu_sc as plsc

NV = 10000
NE = 5000
NEP = 5008
NNZ = 320000
CHUNK = 8000
NGRP = CHUNK // 16
NCHUNK = NNZ // CHUNK
NC = 2
NS = 16
NW = NC * NS


def _mm_body(w_ref, x_ref, b_ref, o_ref, *, dims):
    o_ref[...] = lax.dot_general(
        w_ref[...], x_ref[...], dims, preferred_element_type=jnp.float32
    ) + b_ref[...]


def _mm_xt(W, X, b):
    do = W.shape[1]
    return pl.pallas_call(
        functools.partial(_mm_body, dims=(((0,), (1,)), ((), ()))),
        out_shape=jax.ShapeDtypeStruct((do, X.shape[0]), jnp.float32),
    )(W, X, b.reshape(do, 1))


def _mm_tt(W, Zt, b):
    do = W.shape[1]
    return pl.pallas_call(
        functools.partial(_mm_body, dims=(((0,), (0,)), ((), ()))),
        out_shape=jax.ShapeDtypeStruct((do, Zt.shape[1]), jnp.float32),
    )(W, Zt, b.reshape(do, 1))


def _zero_fill(ref, n16):
    zeros16 = jnp.zeros((16,), jnp.float32)

    def body(j, _):
        ref[pl.ds(j * 16, 16)] = zeros16
        return 0

    lax.fori_loop(0, n16, body, 0, unroll=4)


def _sc_body(*refs, C, first, relu):
    if first:
        yt, vids, eids, zt, rvd_out, red_out = refs[:6]
        rest = refs[6:]
    else:
        yt, vids, eids, rvd_in, red_in, zt = refs[:6]
        rest = refs[6:]
    ab = rest[:C]
    eb = rest[C:2 * C]
    vd, ed, vv, ee = rest[2 * C:]

    wid = lax.axis_index("s") * NC + lax.axis_index("c")
    row0 = wid * C

    ones16 = jnp.full((16,), 1.0, jnp.float32)
    m16 = jnp.full((16,), True)

    for c in range(C):
        pltpu.sync_copy(yt.at[pl.ds((row0 + c) * NV, NV)], ab[c])

    for c in range(C):
        _zero_fill(eb[c], NEP // 16)
    if first:
        _zero_fill(vd, NV // 16)
        _zero_fill(ed, NEP // 16)
    else:
        pltpu.sync_copy(rvd_in, vd)
        pltpu.sync_copy(red_in, ed)

    def pass1_chunk(k, _):
        base = pl.multiple_of(k * CHUNK, 8)
        pltpu.sync_copy(vids.at[pl.ds(base, CHUNK)], vv)
        pltpu.sync_copy(eids.at[pl.ds(base, CHUNK)], ee)

        def grp(g, _):
            off = g * 16
            v16 = vv[pl.ds(off, 16)]
            e16 = ee[pl.ds(off, 16)]
            if first:
                plsc.addupdate_scatter(vd, [v16], ones16, mask=m16)
                plsc.addupdate_scatter(ed, [e16], ones16, mask=m16)
            for c in range(C):
                vals = plsc.load_gather(ab[c], [v16], mask=m16)
                plsc.addupdate_scatter(eb[c], [e16], vals, mask=m16)
            return 0

        lax.fori_loop(0, NGRP, grp, 0, unroll=4)
        return 0

    lax.fori_loop(0, NCHUNK, pass1_chunk, 0)

    if first:
        def rvd_loop(j, _):
            s = pl.ds(j * 16, 16)
            vd[s] = 1.0 / jnp.maximum(vd[s], 1.0)
            return 0
        lax.fori_loop(0, NV // 16, rvd_loop, 0, unroll=4)

        def red_loop(j, _):
            s = pl.ds(j * 16, 16)
            ed[s] = 1.0 / jnp.maximum(ed[s], 1.0)
            return 0
        lax.fori_loop(0, NEP // 16, red_loop, 0, unroll=4)

    def esc(j, _):
        s = pl.ds(j * 16, 16)
        r = ed[s]
        for c in range(C):
            eb[c][s] = eb[c][s] * r
        return 0
    lax.fori_loop(0, NEP // 16, esc, 0, unroll=4)

    for c in range(C):
        _zero_fill(ab[c], NV // 16)

    def pass2_chunk(k, _):
        base = pl.multiple_of(k * CHUNK, 8)
        pltpu.sync_copy(vids.at[pl.ds(base, CHUNK)], vv)
        pltpu.sync_copy(eids.at[pl.ds(base, CHUNK)], ee)

        def grp(g, _):
            off = g * 16
            v16 = vv[pl.ds(off, 16)]
            e16 = ee[pl.ds(off, 16)]
            for c in range(C):
                vals = plsc.load_gather(eb[c], [e16], mask=m16)
                plsc.addupdate_scatter(ab[c], [v16], vals, mask=m16)
            return 0

        lax.fori_loop(0, NGRP, grp, 0, unroll=4)
        return 0

    lax.fori_loop(0, NCHUNK, pass2_chunk, 0)

    def vsc(j, _):
        s = pl.ds(j * 16, 16)
        r = vd[s]
        for c in range(C):
            x = ab[c][s] * r
            if relu:
                x = jnp.maximum(x, 0.0)
            ab[c][s] = x
        return 0
    lax.fori_loop(0, NV // 16, vsc, 0, unroll=4)

    for c in range(C):
        pltpu.sync_copy(ab[c], zt.at[pl.ds((row0 + c) * NV, NV)])

    if first:
        @pl.when(wid == 0)
        def _():
            pltpu.sync_copy(vd, rvd_out)
            pltpu.sync_copy(ed, red_out)


def _make_sc(C, first, relu):
    d = C * NW
    out_type = [jax.ShapeDtypeStruct((d * NV,), jnp.float32)]
    if first:
        out_type += [jax.ShapeDtypeStruct((NV,), jnp.float32),
                     jax.ShapeDtypeStruct((NEP,), jnp.float32)]
    scratch = (
        [pltpu.VMEM((NV,), jnp.float32) for _ in range(C)]
        + [pltpu.VMEM((NEP,), jnp.float32) for _ in range(C)]
        + [
            pltpu.VMEM((NV,), jnp.float32),
            pltpu.VMEM((NEP,), jnp.float32),
            pltpu.VMEM((CHUNK,), jnp.int32),
            pltpu.VMEM((CHUNK,), jnp.int32),
        ]
    )
    mesh = plsc.VectorSubcoreMesh(core_axis_name="c", subcore_axis_name="s")
    return pl.kernel(
        functools.partial(_sc_body, C=C, first=first, relu=relu),
        out_type=out_type,
        mesh=mesh,
        scratch_types=scratch,
        compiler_params=pltpu.CompilerParams(needs_layout_passes=False),
    )


def kernel(X, v_ids, e_ids, W0, b0, W1, b1, W2, b2):
    v32 = v_ids.astype(jnp.int32)
    e32 = e_ids.astype(jnp.int32)

    sc_first = _make_sc(4, True, True)
    sc_mid = _make_sc(4, False, True)
    sc_last = _make_sc(2, False, False)

    y0 = _mm_xt(W0, X, b0)
    z0f, rvd, red = sc_first(y0.reshape(-1), v32, e32)
    z0 = z0f.reshape(128, NV)
    y1 = _mm_tt(W1, z0, b1)
    (z1f,) = sc_mid(y1.reshape(-1), v32, e32, rvd, red)
    z1 = z1f.reshape(128, NV)
    y2 = _mm_tt(W2, z1, b2)
    (z2f,) = sc_last(y2.reshape(-1), v32, e32, rvd, red)
    return z2f.reshape(64, NV).T


# speedup vs baseline: 3.4000x; 1.1548x over previous
import functools

import jax
import jax.numpy as jnp
from jax import lax
from jax.experimental import pallas as pl
from jax.experimental.pallas import tpu as pltpu
from jax.experimental.pallas import tpu_sc as plsc

NV = 10000
NE = 5000
NEP = 5008
NNZ = 320000
CHUNK = 8000
NGRP = CHUNK // 16
NCHUNK = NNZ // CHUNK
NC = 2
NS = 16
NW = NC * NS


def _mm_body(w_ref, x_ref, b_ref, o_ref, *, dims):
    o_ref[...] = lax.dot_general(
        w_ref[...], x_ref[...], dims, preferred_element_type=jnp.float32
    ) + b_ref[...]


def _mm_xt(W, X, b):
    do = W.shape[1]
    return pl.pallas_call(
        functools.partial(_mm_body, dims=(((0,), (1,)), ((), ()))),
        out_shape=jax.ShapeDtypeStruct((do, X.shape[0]), jnp.float32),
    )(W, X, b.reshape(do, 1))


def _mm_tt(W, Zt, b):
    do = W.shape[1]
    return pl.pallas_call(
        functools.partial(_mm_body, dims=(((0,), (0,)), ((), ()))),
        out_shape=jax.ShapeDtypeStruct((do, Zt.shape[1]), jnp.float32),
    )(W, Zt, b.reshape(do, 1))


def _zero_fill(ref, n16):
    zeros16 = jnp.zeros((16,), jnp.float32)

    def body(j, _):
        ref[pl.ds(j * 16, 16)] = zeros16
        return 0

    lax.fori_loop(0, n16, body, 0, unroll=4)


def _sc_body(*refs, C, first, relu):
    if first:
        yt, vids, eids, zt, rvd_out, red_out = refs[:6]
        rest = refs[6:]
    else:
        yt, vids, eids, rvd_in, red_in, zt = refs[:6]
        rest = refs[6:]
    ab = rest[:C]
    eb = rest[C:2 * C]
    vd, ed, vv0, ee0, vv1, ee1, sem0, sem1 = rest[2 * C:]

    wid = lax.axis_index("s") * NC + lax.axis_index("c")
    row0 = wid * C

    ones16 = jnp.full((16,), 1.0, jnp.float32)
    m16 = jnp.full((16,), True)

    for c in range(C):
        pltpu.sync_copy(yt.at[pl.ds((row0 + c) * NV, NV)], ab[c])

    for c in range(C):
        _zero_fill(eb[c], NEP // 16)
    if first:
        _zero_fill(vd, NV // 16)
        _zero_fill(ed, NEP // 16)
    else:
        pltpu.sync_copy(rvd_in, vd)
        pltpu.sync_copy(red_in, ed)

    def _issue(k, vvb, eeb, sem):
        base = pl.multiple_of(jnp.minimum(k * CHUNK, NNZ - CHUNK), 8)
        pltpu.async_copy(vids.at[pl.ds(base, CHUNK)], vvb, sem)
        pltpu.async_copy(eids.at[pl.ds(base, CHUNK)], eeb, sem)

    def _drain(vvb, eeb, sem):
        pltpu.make_async_copy(vids.at[pl.ds(0, CHUNK)], vvb, sem).wait()
        pltpu.make_async_copy(eids.at[pl.ds(0, CHUNK)], eeb, sem).wait()

    def _stream(proc_chunk):
        _issue(0, vv0, ee0, sem0)
        _issue(1, vv1, ee1, sem1)

        def pair(kk, _):
            k0 = 2 * kk
            _drain(vv0, ee0, sem0)
            proc_chunk(vv0, ee0)
            _issue(k0 + 2, vv0, ee0, sem0)
            _drain(vv1, ee1, sem1)
            proc_chunk(vv1, ee1)
            _issue(k0 + 3, vv1, ee1, sem1)
            return 0

        lax.fori_loop(0, NCHUNK // 2, pair, 0)
        _drain(vv0, ee0, sem0)
        _drain(vv1, ee1, sem1)

    def pass1_chunk(vv, ee):
        def grp(g, _):
            off = g * 16
            v16 = vv[pl.ds(off, 16)]
            e16 = ee[pl.ds(off, 16)]
            if first:
                plsc.addupdate_scatter(vd, [v16], ones16, mask=m16)
                plsc.addupdate_scatter(ed, [e16], ones16, mask=m16)
            for c in range(C):
                vals = plsc.load_gather(ab[c], [v16], mask=m16)
                plsc.addupdate_scatter(eb[c], [e16], vals, mask=m16)
            return 0

        lax.fori_loop(0, NGRP, grp, 0, unroll=4)

    _stream(pass1_chunk)

    if first:
        def rvd_loop(j, _):
            s = pl.ds(j * 16, 16)
            vd[s] = 1.0 / jnp.maximum(vd[s], 1.0)
            return 0
        lax.fori_loop(0, NV // 16, rvd_loop, 0, unroll=4)

        def red_loop(j, _):
            s = pl.ds(j * 16, 16)
            ed[s] = 1.0 / jnp.maximum(ed[s], 1.0)
            return 0
        lax.fori_loop(0, NEP // 16, red_loop, 0, unroll=4)

    def esc(j, _):
        s = pl.ds(j * 16, 16)
        r = ed[s]
        for c in range(C):
            eb[c][s] = eb[c][s] * r
        return 0
    lax.fori_loop(0, NEP // 16, esc, 0, unroll=4)

    for c in range(C):
        _zero_fill(ab[c], NV // 16)

    def pass2_chunk(vv, ee):
        def grp(g, _):
            off = g * 16
            v16 = vv[pl.ds(off, 16)]
            e16 = ee[pl.ds(off, 16)]
            for c in range(C):
                vals = plsc.load_gather(eb[c], [e16], mask=m16)
                plsc.addupdate_scatter(ab[c], [v16], vals, mask=m16)
            return 0

        lax.fori_loop(0, NGRP, grp, 0, unroll=4)

    _stream(pass2_chunk)

    def vsc(j, _):
        s = pl.ds(j * 16, 16)
        r = vd[s]
        for c in range(C):
            x = ab[c][s] * r
            if relu:
                x = jnp.maximum(x, 0.0)
            ab[c][s] = x
        return 0
    lax.fori_loop(0, NV // 16, vsc, 0, unroll=4)

    for c in range(C):
        pltpu.sync_copy(ab[c], zt.at[pl.ds((row0 + c) * NV, NV)])

    if first:
        @pl.when(wid == 0)
        def _():
            pltpu.sync_copy(vd, rvd_out)
            pltpu.sync_copy(ed, red_out)


def _make_sc(C, first, relu):
    d = C * NW
    out_type = [jax.ShapeDtypeStruct((d * NV,), jnp.float32)]
    if first:
        out_type += [jax.ShapeDtypeStruct((NV,), jnp.float32),
                     jax.ShapeDtypeStruct((NEP,), jnp.float32)]
    scratch = (
        [pltpu.VMEM((NV,), jnp.float32) for _ in range(C)]
        + [pltpu.VMEM((NEP,), jnp.float32) for _ in range(C)]
        + [
            pltpu.VMEM((NV,), jnp.float32),
            pltpu.VMEM((NEP,), jnp.float32),
            pltpu.VMEM((CHUNK,), jnp.int32),
            pltpu.VMEM((CHUNK,), jnp.int32),
            pltpu.VMEM((CHUNK,), jnp.int32),
            pltpu.VMEM((CHUNK,), jnp.int32),
            pltpu.SemaphoreType.DMA,
            pltpu.SemaphoreType.DMA,
        ]
    )
    mesh = plsc.VectorSubcoreMesh(core_axis_name="c", subcore_axis_name="s")
    return pl.kernel(
        functools.partial(_sc_body, C=C, first=first, relu=relu),
        out_type=out_type,
        mesh=mesh,
        scratch_types=scratch,
        compiler_params=pltpu.CompilerParams(needs_layout_passes=False),
    )


def kernel(X, v_ids, e_ids, W0, b0, W1, b1, W2, b2):
    v32 = v_ids.astype(jnp.int32)
    e32 = e_ids.astype(jnp.int32)

    sc_first = _make_sc(4, True, True)
    sc_mid = _make_sc(4, False, True)
    sc_last = _make_sc(2, False, False)

    y0 = _mm_xt(W0, X, b0)
    z0f, rvd, red = sc_first(y0.reshape(-1), v32, e32)
    z0 = z0f.reshape(128, NV)
    y1 = _mm_tt(W1, z0, b1)
    (z1f,) = sc_mid(y1.reshape(-1), v32, e32, rvd, red)
    z1 = z1f.reshape(128, NV)
    y2 = _mm_tt(W2, z1, b2)
    (z2f,) = sc_last(y2.reshape(-1), v32, e32, rvd, red)
    return z2f.reshape(64, NV).T


# speedup vs baseline: 3.4058x; 1.0017x over previous
import functools

import jax
import jax.numpy as jnp
from jax import lax
from jax.experimental import pallas as pl
from jax.experimental.pallas import tpu as pltpu
from jax.experimental.pallas import tpu_sc as plsc

NV = 10000
NE = 5000
NEP = 5008
NNZ = 320000
CHUNK = 8000
NGRP = CHUNK // 16
NCHUNK = NNZ // CHUNK
NC = 2
NS = 16
NW = NC * NS


def _mm_body(w_ref, x_ref, b_ref, o_ref, *, dims):
    o_ref[...] = lax.dot_general(
        w_ref[...], x_ref[...], dims, preferred_element_type=jnp.float32
    ) + b_ref[...]


def _mm_xt(W, X, b):
    do = W.shape[1]
    return pl.pallas_call(
        functools.partial(_mm_body, dims=(((0,), (1,)), ((), ()))),
        out_shape=jax.ShapeDtypeStruct((do, X.shape[0]), jnp.float32),
    )(W, X, b.reshape(do, 1))


def _mm_tt(W, Zt, b):
    do = W.shape[1]
    return pl.pallas_call(
        functools.partial(_mm_body, dims=(((0,), (0,)), ((), ()))),
        out_shape=jax.ShapeDtypeStruct((do, Zt.shape[1]), jnp.float32),
    )(W, Zt, b.reshape(do, 1))


def _zero_fill(ref, n16):
    zeros16 = jnp.zeros((16,), jnp.float32)

    def body(j, _):
        ref[pl.ds(j * 16, 16)] = zeros16
        return 0

    lax.fori_loop(0, n16, body, 0, unroll=4)


def _sc_body(*refs, C, first, relu):
    if first:
        yt, vids, eids, zt, rvd_out, red_out = refs[:6]
        rest = refs[6:]
    else:
        yt, vids, eids, rvd_in, red_in, zt = refs[:6]
        rest = refs[6:]
    ab = rest[:C]
    eb = rest[C:2 * C]
    vd, ed, vv0, ee0, vv1, ee1, sem0, sem1 = rest[2 * C:]

    wid = lax.axis_index("s") * NC + lax.axis_index("c")
    row0 = wid * C

    ones16 = jnp.full((16,), 1.0, jnp.float32)

    for c in range(C):
        pltpu.sync_copy(yt.at[pl.ds((row0 + c) * NV, NV)], ab[c])

    for c in range(C):
        _zero_fill(eb[c], NEP // 16)
    if first:
        _zero_fill(vd, NV // 16)
        _zero_fill(ed, NEP // 16)
    else:
        pltpu.sync_copy(rvd_in, vd)
        pltpu.sync_copy(red_in, ed)

    def _issue(k, vvb, eeb, sem):
        base = pl.multiple_of(jnp.minimum(k * CHUNK, NNZ - CHUNK), 8)
        pltpu.async_copy(vids.at[pl.ds(base, CHUNK)], vvb, sem)
        pltpu.async_copy(eids.at[pl.ds(base, CHUNK)], eeb, sem)

    def _drain(vvb, eeb, sem):
        pltpu.make_async_copy(vids.at[pl.ds(0, CHUNK)], vvb, sem).wait()
        pltpu.make_async_copy(eids.at[pl.ds(0, CHUNK)], eeb, sem).wait()

    def _stream(proc_chunk):
        _issue(0, vv0, ee0, sem0)
        _issue(1, vv1, ee1, sem1)

        def pair(kk, _):
            k0 = 2 * kk
            _drain(vv0, ee0, sem0)
            proc_chunk(vv0, ee0)
            _issue(k0 + 2, vv0, ee0, sem0)
            _drain(vv1, ee1, sem1)
            proc_chunk(vv1, ee1)
            _issue(k0 + 3, vv1, ee1, sem1)
            return 0

        lax.fori_loop(0, NCHUNK // 2, pair, 0)
        _drain(vv0, ee0, sem0)
        _drain(vv1, ee1, sem1)

    def pass1_chunk(vv, ee):
        def grp(g, _):
            off = g * 16
            v16 = vv[pl.ds(off, 16)]
            e16 = ee[pl.ds(off, 16)]
            if first:
                plsc.addupdate_scatter(vd, [v16], ones16)
                plsc.addupdate_scatter(ed, [e16], ones16)
            for c in range(C):
                vals = plsc.load_gather(ab[c], [v16])
                plsc.addupdate_scatter(eb[c], [e16], vals)
            return 0

        lax.fori_loop(0, NGRP, grp, 0, unroll=8)

    _stream(pass1_chunk)

    if first:
        def rvd_loop(j, _):
            s = pl.ds(j * 16, 16)
            vd[s] = 1.0 / jnp.maximum(vd[s], 1.0)
            return 0
        lax.fori_loop(0, NV // 16, rvd_loop, 0, unroll=4)

        def red_loop(j, _):
            s = pl.ds(j * 16, 16)
            ed[s] = 1.0 / jnp.maximum(ed[s], 1.0)
            return 0
        lax.fori_loop(0, NEP // 16, red_loop, 0, unroll=4)

    def esc(j, _):
        s = pl.ds(j * 16, 16)
        r = ed[s]
        for c in range(C):
            eb[c][s] = eb[c][s] * r
        return 0
    lax.fori_loop(0, NEP // 16, esc, 0, unroll=4)

    for c in range(C):
        _zero_fill(ab[c], NV // 16)

    def pass2_chunk(vv, ee):
        def grp(g, _):
            off = g * 16
            v16 = vv[pl.ds(off, 16)]
            e16 = ee[pl.ds(off, 16)]
            for c in range(C):
                vals = plsc.load_gather(eb[c], [e16])
                plsc.addupdate_scatter(ab[c], [v16], vals)
            return 0

        lax.fori_loop(0, NGRP, grp, 0, unroll=8)

    _stream(pass2_chunk)

    def vsc(j, _):
        s = pl.ds(j * 16, 16)
        r = vd[s]
        for c in range(C):
            x = ab[c][s] * r
            if relu:
                x = jnp.maximum(x, 0.0)
            ab[c][s] = x
        return 0
    lax.fori_loop(0, NV // 16, vsc, 0, unroll=4)

    for c in range(C):
        pltpu.sync_copy(ab[c], zt.at[pl.ds((row0 + c) * NV, NV)])

    if first:
        @pl.when(wid == 0)
        def _():
            pltpu.sync_copy(vd, rvd_out)
            pltpu.sync_copy(ed, red_out)


def _make_sc(C, first, relu):
    d = C * NW
    out_type = [jax.ShapeDtypeStruct((d * NV,), jnp.float32)]
    if first:
        out_type += [jax.ShapeDtypeStruct((NV,), jnp.float32),
                     jax.ShapeDtypeStruct((NEP,), jnp.float32)]
    scratch = (
        [pltpu.VMEM((NV,), jnp.float32) for _ in range(C)]
        + [pltpu.VMEM((NEP,), jnp.float32) for _ in range(C)]
        + [
            pltpu.VMEM((NV,), jnp.float32),
            pltpu.VMEM((NEP,), jnp.float32),
            pltpu.VMEM((CHUNK,), jnp.int32),
            pltpu.VMEM((CHUNK,), jnp.int32),
            pltpu.VMEM((CHUNK,), jnp.int32),
            pltpu.VMEM((CHUNK,), jnp.int32),
            pltpu.SemaphoreType.DMA,
            pltpu.SemaphoreType.DMA,
        ]
    )
    mesh = plsc.VectorSubcoreMesh(core_axis_name="c", subcore_axis_name="s")
    return pl.kernel(
        functools.partial(_sc_body, C=C, first=first, relu=relu),
        out_type=out_type,
        mesh=mesh,
        scratch_types=scratch,
        compiler_params=pltpu.CompilerParams(needs_layout_passes=False),
    )


def kernel(X, v_ids, e_ids, W0, b0, W1, b1, W2, b2):
    v32 = v_ids.astype(jnp.int32)
    e32 = e_ids.astype(jnp.int32)

    sc_first = _make_sc(4, True, True)
    sc_mid = _make_sc(4, False, True)
    sc_last = _make_sc(2, False, False)

    y0 = _mm_xt(W0, X, b0)
    z0f, rvd, red = sc_first(y0.reshape(-1), v32, e32)
    z0 = z0f.reshape(128, NV)
    y1 = _mm_tt(W1, z0, b1)
    (z1f,) = sc_mid(y1.reshape(-1), v32, e32, rvd, red)
    z1 = z1f.reshape(128, NV)
    y2 = _mm_tt(W2, z1, b2)
    (z2f,) = sc_last(y2.reshape(-1), v32, e32, rvd, red)
    return z2f.reshape(64, NV).T


# speedup vs baseline: 4.2890x; 1.2593x over previous
import functools

import jax
import jax.numpy as jnp
from jax import lax
from jax.experimental import pallas as pl
from jax.experimental.pallas import tpu as pltpu
from jax.experimental.pallas import tpu_sc as plsc

NV = 10000
NE = 5000
NEP = 5008
NNZ = 320000
CHUNK = 4000
NGRP = CHUNK // 16
NCHUNK = NNZ // CHUNK
NC = 2
NS = 16
NW = NC * NS


def _mm_body(w_ref, x_ref, b_ref, o_ref, *, dims):
    o_ref[...] = lax.dot_general(
        w_ref[...], x_ref[...], dims, preferred_element_type=jnp.float32
    ) + b_ref[...]


def _mm_xt(W, X, b):
    do = W.shape[1]
    return pl.pallas_call(
        functools.partial(_mm_body, dims=(((0,), (1,)), ((), ()))),
        out_shape=jax.ShapeDtypeStruct((do, X.shape[0]), jnp.float32),
    )(W, X, b.reshape(do, 1))


def _mm_tt(W, Zt, b):
    do = W.shape[1]
    return pl.pallas_call(
        functools.partial(_mm_body, dims=(((0,), (0,)), ((), ()))),
        out_shape=jax.ShapeDtypeStruct((do, Zt.shape[1]), jnp.float32),
    )(W, Zt, b.reshape(do, 1))


def _zero_fill(ref, n16):
    zeros16 = jnp.zeros((16,), jnp.float32)

    def body(j, _):
        ref[pl.ds(j * 16, 16)] = zeros16
        return 0

    lax.fori_loop(0, n16, body, 0, unroll=4)


def _sc_body(*refs, C, first, relu):
    if first:
        yt, vids, eids, zt, rvd_out, red_out = refs[:6]
        rest = refs[6:]
    else:
        yt, vids, eids, rvd_in, red_in, zt = refs[:6]
        rest = refs[6:]
    npair = C // 2
    ab = rest[:C]
    eb = rest[C:2 * C]
    pxb = rest[2 * C:2 * C + npair]
    peb = rest[2 * C + npair:2 * C + 2 * npair]
    vd, ed, vv0, ee0, vv1, ee1, sem0, sem1 = rest[2 * C + 2 * npair:]

    wid = lax.axis_index("s") * NC + lax.axis_index("c")
    row0 = wid * C

    ones16 = jnp.full((16,), 1.0, jnp.float32)

    for c in range(C):
        pltpu.sync_copy(yt.at[pl.ds((row0 + c) * NV, NV)], ab[c])

    def pkx(j, _):
        s = pl.ds(j * 16, 16)
        for p in range(npair):
            w = plsc.pack(ab[2 * p][s], ab[2 * p + 1][s],
                          format=plsc.PackFormat.INTERLEAVED)
            pxb[p][s] = plsc.bitcast(w, jnp.float32)
        return 0
    lax.fori_loop(0, NV // 16, pkx, 0, unroll=4)

    for c in range(C):
        _zero_fill(eb[c], NEP // 16)
    if first:
        _zero_fill(vd, NV // 16)
        _zero_fill(ed, NEP // 16)
    else:
        pltpu.sync_copy(rvd_in, vd)
        pltpu.sync_copy(red_in, ed)

    def _issue(k, vvb, eeb, sem):
        base = pl.multiple_of(jnp.minimum(k * CHUNK, NNZ - CHUNK), 8)
        pltpu.async_copy(vids.at[pl.ds(base, CHUNK)], vvb, sem)
        pltpu.async_copy(eids.at[pl.ds(base, CHUNK)], eeb, sem)

    def _drain(vvb, eeb, sem):
        pltpu.make_async_copy(vids.at[pl.ds(0, CHUNK)], vvb, sem).wait()
        pltpu.make_async_copy(eids.at[pl.ds(0, CHUNK)], eeb, sem).wait()

    def _stream(proc_chunk):
        _issue(0, vv0, ee0, sem0)
        _issue(1, vv1, ee1, sem1)

        def pair(kk, _):
            k0 = 2 * kk
            _drain(vv0, ee0, sem0)
            proc_chunk(vv0, ee0)
            _issue(k0 + 2, vv0, ee0, sem0)
            _drain(vv1, ee1, sem1)
            proc_chunk(vv1, ee1)
            _issue(k0 + 3, vv1, ee1, sem1)
            return 0

        lax.fori_loop(0, NCHUNK // 2, pair, 0)
        _drain(vv0, ee0, sem0)
        _drain(vv1, ee1, sem1)

    def pass1_chunk(vv, ee):
        def grp(g, _):
            off = g * 16
            v16 = vv[pl.ds(off, 16)]
            e16 = ee[pl.ds(off, 16)]
            if first:
                plsc.addupdate_scatter(vd, [v16], ones16)
                plsc.addupdate_scatter(ed, [e16], ones16)
            for p in range(npair):
                w = plsc.load_gather(pxb[p], [v16])
                a, b = plsc.unpack(plsc.bitcast(w, jnp.bfloat16),
                                   format=plsc.PackFormat.INTERLEAVED)
                plsc.addupdate_scatter(eb[2 * p], [e16], a)
                plsc.addupdate_scatter(eb[2 * p + 1], [e16], b)
            return 0

        lax.fori_loop(0, NGRP, grp, 0, unroll=8)

    _stream(pass1_chunk)

    if first:
        def rvd_loop(j, _):
            s = pl.ds(j * 16, 16)
            vd[s] = 1.0 / jnp.maximum(vd[s], 1.0)
            return 0
        lax.fori_loop(0, NV // 16, rvd_loop, 0, unroll=4)

        def red_loop(j, _):
            s = pl.ds(j * 16, 16)
            ed[s] = 1.0 / jnp.maximum(ed[s], 1.0)
            return 0
        lax.fori_loop(0, NEP // 16, red_loop, 0, unroll=4)

    def esc(j, _):
        s = pl.ds(j * 16, 16)
        r = ed[s]
        for p in range(npair):
            a = eb[2 * p][s] * r
            b = eb[2 * p + 1][s] * r
            w = plsc.pack(a, b, format=plsc.PackFormat.INTERLEAVED)
            peb[p][s] = plsc.bitcast(w, jnp.float32)
        return 0
    lax.fori_loop(0, NEP // 16, esc, 0, unroll=4)

    for c in range(C):
        _zero_fill(ab[c], NV // 16)

    def pass2_chunk(vv, ee):
        def grp(g, _):
            off = g * 16
            v16 = vv[pl.ds(off, 16)]
            e16 = ee[pl.ds(off, 16)]
            for p in range(npair):
                w = plsc.load_gather(peb[p], [e16])
                a, b = plsc.unpack(plsc.bitcast(w, jnp.bfloat16),
                                   format=plsc.PackFormat.INTERLEAVED)
                plsc.addupdate_scatter(ab[2 * p], [v16], a)
                plsc.addupdate_scatter(ab[2 * p + 1], [v16], b)
            return 0

        lax.fori_loop(0, NGRP, grp, 0, unroll=8)

    _stream(pass2_chunk)

    def vsc(j, _):
        s = pl.ds(j * 16, 16)
        r = vd[s]
        for c in range(C):
            x = ab[c][s] * r
            if relu:
                x = jnp.maximum(x, 0.0)
            ab[c][s] = x
        return 0
    lax.fori_loop(0, NV // 16, vsc, 0, unroll=4)

    for c in range(C):
        pltpu.sync_copy(ab[c], zt.at[pl.ds((row0 + c) * NV, NV)])

    if first:
        @pl.when(wid == 0)
        def _():
            pltpu.sync_copy(vd, rvd_out)
            pltpu.sync_copy(ed, red_out)


def _make_sc(C, first, relu):
    d = C * NW
    out_type = [jax.ShapeDtypeStruct((d * NV,), jnp.float32)]
    if first:
        out_type += [jax.ShapeDtypeStruct((NV,), jnp.float32),
                     jax.ShapeDtypeStruct((NEP,), jnp.float32)]
    scratch = (
        [pltpu.VMEM((NV,), jnp.float32) for _ in range(C)]
        + [pltpu.VMEM((NEP,), jnp.float32) for _ in range(C)]
        + [pltpu.VMEM((NV,), jnp.float32) for _ in range(C // 2)]
        + [pltpu.VMEM((NEP,), jnp.float32) for _ in range(C // 2)]
        + [
            pltpu.VMEM((NV,), jnp.float32),
            pltpu.VMEM((NEP,), jnp.float32),
            pltpu.VMEM((CHUNK,), jnp.int32),
            pltpu.VMEM((CHUNK,), jnp.int32),
            pltpu.VMEM((CHUNK,), jnp.int32),
            pltpu.VMEM((CHUNK,), jnp.int32),
            pltpu.SemaphoreType.DMA,
            pltpu.SemaphoreType.DMA,
        ]
    )
    mesh = plsc.VectorSubcoreMesh(core_axis_name="c", subcore_axis_name="s")
    return pl.kernel(
        functools.partial(_sc_body, C=C, first=first, relu=relu),
        out_type=out_type,
        mesh=mesh,
        scratch_types=scratch,
        compiler_params=pltpu.CompilerParams(needs_layout_passes=False),
    )


def kernel(X, v_ids, e_ids, W0, b0, W1, b1, W2, b2):
    v32 = v_ids.astype(jnp.int32)
    e32 = e_ids.astype(jnp.int32)

    sc_first = _make_sc(4, True, True)
    sc_mid = _make_sc(4, False, True)
    sc_last = _make_sc(2, False, False)

    y0 = _mm_xt(W0, X, b0)
    z0f, rvd, red = sc_first(y0.reshape(-1), v32, e32)
    z0 = z0f.reshape(128, NV)
    y1 = _mm_tt(W1, z0, b1)
    (z1f,) = sc_mid(y1.reshape(-1), v32, e32, rvd, red)
    z1 = z1f.reshape(128, NV)
    y2 = _mm_tt(W2, z1, b2)
    (z2f,) = sc_last(y2.reshape(-1), v32, e32, rvd, red)
    return z2f.reshape(64, NV).T


# speedup vs baseline: 6.4577x; 1.5056x over previous
import functools

import jax
import jax.numpy as jnp
from jax import lax
from jax.experimental import pallas as pl
from jax.experimental.pallas import tpu as pltpu
from jax.experimental.pallas import tpu_sc as plsc

NV = 10000
NE = 5000
NEP = 5008
NNZ = 320000
CHUNK = 4000
NGRP = CHUNK // 16
NCHUNK = NNZ // CHUNK
NC = 2
NS = 16
NW = NC * NS


def _mm_body(w_ref, x_ref, b_ref, o_ref, *, dims):
    o_ref[...] = lax.dot_general(
        w_ref[...], x_ref[...], dims, preferred_element_type=jnp.float32
    ) + b_ref[...]


def _mm_xt(W, X, b):
    do = W.shape[1]
    return pl.pallas_call(
        functools.partial(_mm_body, dims=(((0,), (1,)), ((), ()))),
        out_shape=jax.ShapeDtypeStruct((do, X.shape[0]), jnp.float32),
    )(W, X, b.reshape(do, 1))


def _mm_tt(W, Zt, b):
    do = W.shape[1]
    return pl.pallas_call(
        functools.partial(_mm_body, dims=(((0,), (0,)), ((), ()))),
        out_shape=jax.ShapeDtypeStruct((do, Zt.shape[1]), jnp.float32),
    )(W, Zt, b.reshape(do, 1))


def _zero_fill(ref, n16):
    zeros16 = jnp.zeros((16,), jnp.float32)

    def body(j, _):
        ref[pl.ds(j * 16, 16)] = zeros16
        return 0

    lax.fori_loop(0, n16, body, 0, unroll=4)


def _sc_body(*refs, C, first, relu):
    if first:
        yt, vids, eids, zt, rvd_out, red_out = refs[:6]
        rest = refs[6:]
    else:
        yt, vids, eids, rvd_in, red_in, zt = refs[:6]
        rest = refs[6:]
    npair = C // 2
    ab = rest[:C]
    eb = rest[C:2 * C]
    pxb = rest[2 * C:2 * C + npair]
    peb = rest[2 * C + npair:2 * C + 2 * npair]
    vd, ed, vv0, ee0, vv1, ee1, sem0, sem1 = rest[2 * C + 2 * npair:]

    wid = lax.axis_index("s") * NC + lax.axis_index("c")
    row0 = wid * C

    ones16 = jnp.full((16,), 1.0, jnp.float32)

    for c in range(C):
        pltpu.sync_copy(yt.at[pl.ds((row0 + c) * NV, NV)], ab[c])

    def pkx(j, _):
        s = pl.ds(j * 16, 16)
        for p in range(npair):
            w = plsc.pack(ab[2 * p][s], ab[2 * p + 1][s],
                          format=plsc.PackFormat.INTERLEAVED)
            pxb[p][s] = plsc.bitcast(w, jnp.float32)
        return 0
    lax.fori_loop(0, NV // 16, pkx, 0, unroll=4)

    for c in range(C):
        _zero_fill(eb[c], NEP // 16)
    if first:
        _zero_fill(vd, NV // 16)
        _zero_fill(ed, NEP // 16)
    else:
        pltpu.sync_copy(rvd_in, vd)
        pltpu.sync_copy(red_in, ed)

    def _issue(k, vvb, eeb, sem):
        base = pl.multiple_of(jnp.minimum(k * CHUNK, NNZ - CHUNK), 8)
        pltpu.async_copy(vids.at[pl.ds(base, CHUNK)], vvb, sem)
        pltpu.async_copy(eids.at[pl.ds(base, CHUNK)], eeb, sem)

    def _drain(vvb, eeb, sem):
        pltpu.make_async_copy(vids.at[pl.ds(0, CHUNK)], vvb, sem).wait()
        pltpu.make_async_copy(eids.at[pl.ds(0, CHUNK)], eeb, sem).wait()

    def _stream(proc_chunk):
        _issue(0, vv0, ee0, sem0)
        _issue(1, vv1, ee1, sem1)

        def pair(kk, _):
            k0 = 2 * kk
            _drain(vv0, ee0, sem0)
            proc_chunk(vv0, ee0)
            _issue(k0 + 2, vv0, ee0, sem0)
            _drain(vv1, ee1, sem1)
            proc_chunk(vv1, ee1)
            _issue(k0 + 3, vv1, ee1, sem1)
            return 0

        lax.fori_loop(0, NCHUNK // 2, pair, 0)
        _drain(vv0, ee0, sem0)
        _drain(vv1, ee1, sem1)

    def pass1_chunk(vv, ee):
        v0 = vv[pl.ds(0, 16)]
        e0 = ee[pl.ds(0, 16)]
        w0 = [plsc.load_gather(pxb[p], [v0]) for p in range(npair)]

        def grp(g, carry):
            v16, e16 = carry[0], carry[1]
            ws = carry[2:]
            offn = pl.ds(jnp.minimum((g + 1) * 16, CHUNK - 16), 16)
            vn = vv[offn]
            en = ee[offn]
            wn = [plsc.load_gather(pxb[p], [vn]) for p in range(npair)]
            if first:
                plsc.addupdate_scatter(vd, [v16], ones16)
                plsc.addupdate_scatter(ed, [e16], ones16)
            for p in range(npair):
                a, b = plsc.unpack(plsc.bitcast(ws[p], jnp.bfloat16),
                                   format=plsc.PackFormat.INTERLEAVED)
                plsc.addupdate_scatter(eb[2 * p], [e16], a)
                plsc.addupdate_scatter(eb[2 * p + 1], [e16], b)
            return (vn, en, *wn)

        lax.fori_loop(0, NGRP, grp, (v0, e0, *w0), unroll=8)

    _stream(pass1_chunk)

    if first:
        def rvd_loop(j, _):
            s = pl.ds(j * 16, 16)
            vd[s] = 1.0 / jnp.maximum(vd[s], 1.0)
            return 0
        lax.fori_loop(0, NV // 16, rvd_loop, 0, unroll=4)

        def red_loop(j, _):
            s = pl.ds(j * 16, 16)
            ed[s] = 1.0 / jnp.maximum(ed[s], 1.0)
            return 0
        lax.fori_loop(0, NEP // 16, red_loop, 0, unroll=4)

    def esc(j, _):
        s = pl.ds(j * 16, 16)
        r = ed[s]
        for p in range(npair):
            a = eb[2 * p][s] * r
            b = eb[2 * p + 1][s] * r
            w = plsc.pack(a, b, format=plsc.PackFormat.INTERLEAVED)
            peb[p][s] = plsc.bitcast(w, jnp.float32)
        return 0
    lax.fori_loop(0, NEP // 16, esc, 0, unroll=4)

    for c in range(C):
        _zero_fill(ab[c], NV // 16)

    def pass2_chunk(vv, ee):
        v0 = vv[pl.ds(0, 16)]
        e0 = ee[pl.ds(0, 16)]
        w0 = [plsc.load_gather(peb[p], [e0]) for p in range(npair)]

        def grp(g, carry):
            v16, e16 = carry[0], carry[1]
            ws = carry[2:]
            offn = pl.ds(jnp.minimum((g + 1) * 16, CHUNK - 16), 16)
            vn = vv[offn]
            en = ee[offn]
            wn = [plsc.load_gather(peb[p], [en]) for p in range(npair)]
            for p in range(npair):
                a, b = plsc.unpack(plsc.bitcast(ws[p], jnp.bfloat16),
                                   format=plsc.PackFormat.INTERLEAVED)
                plsc.addupdate_scatter(ab[2 * p], [v16], a)
                plsc.addupdate_scatter(ab[2 * p + 1], [v16], b)
            return (vn, en, *wn)

        lax.fori_loop(0, NGRP, grp, (v0, e0, *w0), unroll=8)

    _stream(pass2_chunk)

    def vsc(j, _):
        s = pl.ds(j * 16, 16)
        r = vd[s]
        for c in range(C):
            x = ab[c][s] * r
            if relu:
                x = jnp.maximum(x, 0.0)
            ab[c][s] = x
        return 0
    lax.fori_loop(0, NV // 16, vsc, 0, unroll=4)

    for c in range(C):
        pltpu.sync_copy(ab[c], zt.at[pl.ds((row0 + c) * NV, NV)])

    if first:
        @pl.when(wid == 0)
        def _():
            pltpu.sync_copy(vd, rvd_out)
            pltpu.sync_copy(ed, red_out)


def _make_sc(C, first, relu):
    d = C * NW
    out_type = [jax.ShapeDtypeStruct((d * NV,), jnp.float32)]
    if first:
        out_type += [jax.ShapeDtypeStruct((NV,), jnp.float32),
                     jax.ShapeDtypeStruct((NEP,), jnp.float32)]
    scratch = (
        [pltpu.VMEM((NV,), jnp.float32) for _ in range(C)]
        + [pltpu.VMEM((NEP,), jnp.float32) for _ in range(C)]
        + [pltpu.VMEM((NV,), jnp.float32) for _ in range(C // 2)]
        + [pltpu.VMEM((NEP,), jnp.float32) for _ in range(C // 2)]
        + [
            pltpu.VMEM((NV,), jnp.float32),
            pltpu.VMEM((NEP,), jnp.float32),
            pltpu.VMEM((CHUNK,), jnp.int32),
            pltpu.VMEM((CHUNK,), jnp.int32),
            pltpu.VMEM((CHUNK,), jnp.int32),
            pltpu.VMEM((CHUNK,), jnp.int32),
            pltpu.SemaphoreType.DMA,
            pltpu.SemaphoreType.DMA,
        ]
    )
    mesh = plsc.VectorSubcoreMesh(core_axis_name="c", subcore_axis_name="s")
    return pl.kernel(
        functools.partial(_sc_body, C=C, first=first, relu=relu),
        out_type=out_type,
        mesh=mesh,
        scratch_types=scratch,
        compiler_params=pltpu.CompilerParams(needs_layout_passes=False),
    )


def kernel(X, v_ids, e_ids, W0, b0, W1, b1, W2, b2):
    v32 = v_ids.astype(jnp.int32)
    e32 = e_ids.astype(jnp.int32)

    sc_first = _make_sc(4, True, True)
    sc_mid = _make_sc(4, False, True)
    sc_last = _make_sc(2, False, False)

    y0 = _mm_xt(W0, X, b0)
    z0f, rvd, red = sc_first(y0.reshape(-1), v32, e32)
    z0 = z0f.reshape(128, NV)
    y1 = _mm_tt(W1, z0, b1)
    (z1f,) = sc_mid(y1.reshape(-1), v32, e32, rvd, red)
    z1 = z1f.reshape(128, NV)
    y2 = _mm_tt(W2, z1, b2)
    (z2f,) = sc_last(y2.reshape(-1), v32, e32, rvd, red)
    return z2f.reshape(64, NV).T
